# baseline (speedup 1.0000x reference)
import functools

import jax
import jax.numpy as jnp
from jax import lax
from jax.experimental import pallas as pl
from jax.experimental.pallas import tpu as pltpu
from jax.experimental.pallas import tpu_sc as plsc

H = 128
NUM_TASKS = 128
G = 128

NC = 2
NS = 16
LANES = 16

NPAD = 50688
B = 512
NB = NPAD // B
CB = 2048
KG = 128
FB = CB + KG
BIGDST = 1 << 28


def _ln(x, g, be):
    m = jnp.mean(x, axis=-1, keepdims=True)
    v = jnp.mean((x - m) ** 2, axis=-1, keepdims=True)
    return (x - m) / jnp.sqrt(v + 1e-5) * g + be


def _mp_body(D, R, EC, nchunk, npasses,
             h_hbm, src_hbm, dst_hbm, a0_hbm, a1_hbm, a2_hbm, wt_hbm, be_hbm,
             agg_hbm,
             acc, srcb, dstb, a0b, a1b, a2b, fsrc, fdst, fa0, fa1, fa2,
             rows, msg, dkidx, wtv, bev):
    c = lax.axis_index("c")
    s = lax.axis_index("s")
    nv = D // LANES
    stripe = R // NS

    pltpu.sync_copy(wt_hbm, wtv)
    pltpu.sync_copy(be_hbm, bev)

    zi = jnp.zeros((LANES,), jnp.int32)

    def zidx(i, _):
        fsrc[pl.ds(i * LANES, LANES)] = zi
        fdst[pl.ds(i * LANES, LANES)] = zi
        return 0
    lax.fori_loop(0, FB // LANES, zidx, 0)

    tile_e0 = s * EC

    for p in range(npasses):
        lo = (c * npasses + p) * R
        hi = lo + R

        zf = jnp.zeros((LANES,), jnp.float32)

        def zmsg(i, _):
            msg[i // nv, pl.ds((i % nv) * LANES, LANES)] = zf
            return 0
        lax.fori_loop(0, KG * nv, zmsg, 0)

        row0 = s * stripe
        nfull, rem = stripe // KG, stripe % KG
        for u in range(nfull):
            pltpu.sync_copy(msg, acc.at[pl.ds(row0 + u * KG, KG)])
        if rem:
            pltpu.sync_copy(msg.at[pl.ds(0, rem)],
                            acc.at[pl.ds(row0 + nfull * KG, rem)])
        plsc.subcore_barrier()

        def chunk_body(t, _):
            e0 = tile_e0 + t * CB
            pltpu.sync_copy(src_hbm.at[pl.ds(e0, CB)], srcb)
            pltpu.sync_copy(dst_hbm.at[pl.ds(e0, CB)], dstb)
            pltpu.sync_copy(a0_hbm.at[pl.ds(e0, CB)], a0b)
            pltpu.sync_copy(a1_hbm.at[pl.ds(e0, CB)], a1b)
            pltpu.sync_copy(a2_hbm.at[pl.ds(e0, CB)], a2b)

            def scan_body(i, cnt):
                dv = dstb[pl.ds(i * LANES, LANES)]
                m = (dv >= lo) & (dv < hi)
                plsc.store_compressed(fdst.at[pl.ds(cnt, LANES)], dv - lo,
                                      mask=m)
                plsc.store_compressed(fsrc.at[pl.ds(cnt, LANES)],
                                      srcb[pl.ds(i * LANES, LANES)], mask=m)
                plsc.store_compressed(fa0.at[pl.ds(cnt, LANES)],
                                      a0b[pl.ds(i * LANES, LANES)], mask=m)
                plsc.store_compressed(fa1.at[pl.ds(cnt, LANES)],
                                      a1b[pl.ds(i * LANES, LANES)], mask=m)
                plsc.store_compressed(fa2.at[pl.ds(cnt, LANES)],
                                      a2b[pl.ds(i * LANES, LANES)], mask=m)
                return cnt + jnp.sum(m.astype(jnp.int32))

            cnt = lax.fori_loop(0, CB // LANES, scan_body, 0)
            ng = (cnt + KG - 1) // KG

            def flush(g, _):
                base = g * KG
                pltpu.sync_copy(h_hbm.at[fsrc.at[pl.ds(base, KG)]], rows)

                def row16(jj, _):
                    rbase = jj * LANES
                    a0v = fa0[pl.ds(base + rbase, LANES)]
                    a1v = fa1[pl.ds(base + rbase, LANES)]
                    a2v = fa2[pl.ds(base + rbase, LANES)]
                    pos = base + rbase + lax.iota(jnp.int32, LANES)
                    vmv = jnp.where(pos < cnt, 1.0, 0.0).astype(jnp.float32)
                    for j in range(LANES):
                        row = rbase + j
                        a0s, a1s, a2s, vm = a0v[j], a1v[j], a2v[j], vmv[j]
                        for r in range(nv):
                            dsr = pl.ds(r * LANES, LANES)
                            ev = (a0s * wtv[0, dsr] + a1s * wtv[1, dsr]
                                  + a2s * wtv[2, dsr] + bev[dsr])
                            msg[row, dsr] = (
                                jnp.maximum(rows[row, dsr] + ev, 0.0) * vm)
                    return 0
                lax.fori_loop(0, KG // LANES, row16, 0)

                for u in range(KG // LANES):
                    dkidx[pl.ds(u * LANES, LANES)] = (
                        fdst[pl.ds(base + u * LANES, LANES)])
                pltpu.sync_copy(msg, acc.at[dkidx], add=True)
                return 0

            lax.fori_loop(0, ng, flush, 0)
            return 0

        lax.fori_loop(0, nchunk, chunk_body, 0)
        plsc.subcore_barrier()

        for u in range(nfull):
            pltpu.sync_copy(acc.at[pl.ds(row0 + u * KG, KG)], rows)
            pltpu.sync_copy(rows, agg_hbm.at[pl.ds(lo + row0 + u * KG, KG)])
        if rem:
            pltpu.sync_copy(acc.at[pl.ds(row0 + nfull * KG, rem)],
                            rows.at[pl.ds(0, rem)])
            pltpu.sync_copy(rows.at[pl.ds(0, rem)],
                            agg_hbm.at[pl.ds(lo + row0 + nfull * KG, rem)])
        plsc.subcore_barrier()


def _mp(h, srcp, dstp, a0p, a1p, a2p, wt, be, D, npasses):
    EP = srcp.shape[0]
    EC = EP // NS
    nchunk = EC // CB
    R = NPAD // (NC * npasses)
    mesh = plsc.VectorSubcoreMesh(core_axis_name="c", subcore_axis_name="s")
    body = functools.partial(_mp_body, D, R, EC, nchunk, npasses)
    f = pl.kernel(
        body,
        out_type=jax.ShapeDtypeStruct((NPAD, D), jnp.float32),
        mesh=mesh,
        compiler_params=pltpu.CompilerParams(needs_layout_passes=False),
        scratch_types=[
            pltpu.VMEM_SHARED((R, D), jnp.float32),
            pltpu.VMEM((CB,), jnp.int32), pltpu.VMEM((CB,), jnp.int32),
            pltpu.VMEM((CB,), jnp.float32), pltpu.VMEM((CB,), jnp.float32),
            pltpu.VMEM((CB,), jnp.float32),
            pltpu.VMEM((FB,), jnp.int32), pltpu.VMEM((FB,), jnp.int32),
            pltpu.VMEM((FB,), jnp.float32), pltpu.VMEM((FB,), jnp.float32),
            pltpu.VMEM((FB,), jnp.float32),
            pltpu.VMEM((KG, D), jnp.float32), pltpu.VMEM((KG, D), jnp.float32),
            pltpu.VMEM((KG,), jnp.int32),
            pltpu.VMEM((3, D), jnp.float32), pltpu.VMEM((D,), jnp.float32),
        ],
    )
    return f(h, srcp, dstp, a0p, a1p, a2p, wt, be)


TROWS = 136


def _pool_body(h_hbm, gate_hbm, batch_hbm, maxp_hbm, gmaxp_hbm,
               mtab, gtab, hb, gb, bb, red, gred, gbb, shm, gsh):
    c = lax.axis_index("c")
    s = lax.axis_index("s")
    wid = c * NS + s
    SR = NPAD // (NC * NS)
    CR = 144
    nch = SR // CR
    neg = jnp.full((LANES,), -3.0e38, jnp.float32)

    def ztab(i, _):
        mtab[i // 8, pl.ds((i % 8) * LANES, LANES)] = neg
        return 0
    lax.fori_loop(0, TROWS * 8, ztab, 0)

    def zgtab(i, _):
        gtab[i, pl.ds(0, LANES)] = neg
        return 0
    lax.fori_loop(0, TROWS, zgtab, 0)

    r0 = wid * SR

    def chunk(t, _):
        base = r0 + t * CR
        pltpu.sync_copy(h_hbm.at[pl.ds(base, CR)], hb)
        pltpu.sync_copy(gate_hbm.at[pl.ds(base, CR)], gb)
        pltpu.sync_copy(batch_hbm.at[pl.ds(base, CR)], bb)

        def row16(jj, _):
            rbase = jj * LANES
            bv = bb[pl.ds(rbase, LANES)]
            gv = gb[pl.ds(rbase, LANES)]
            for j in range(LANES):
                b = bv[j]
                row = rbase + j
                for r in range(8):
                    dsr = pl.ds(r * LANES, LANES)
                    mtab[b, dsr] = jnp.maximum(mtab[b, dsr], hb[row, dsr])
                gtab[b, pl.ds(0, LANES)] = jnp.maximum(
                    gtab[b, pl.ds(0, LANES)], gv[j])
            return 0
        lax.fori_loop(0, CR // LANES, row16, 0)
        return 0

    lax.fori_loop(0, nch, chunk, 0)

    pltpu.sync_copy(mtab, shm.at[s])
    pltpu.sync_copy(gtab, gsh.at[s])
    plsc.subcore_barrier()

    g0 = s * 8
    for t2 in range(NS):
        if t2 == 0:
            pltpu.sync_copy(shm.at[t2, pl.ds(g0, 8)], red)
            pltpu.sync_copy(gsh.at[t2, pl.ds(g0, 8)], gred)
        else:
            pltpu.sync_copy(shm.at[t2, pl.ds(g0, 8)], hb.at[pl.ds(0, 8)])
            pltpu.sync_copy(gsh.at[t2, pl.ds(g0, 8)], gbb)
            for j in range(8):
                for r in range(8):
                    dsr = pl.ds(r * LANES, LANES)
                    red[j, dsr] = jnp.maximum(red[j, dsr], hb[j, dsr])
                gred[j, pl.ds(0, LANES)] = jnp.maximum(
                    gred[j, pl.ds(0, LANES)], gbb[j, pl.ds(0, LANES)])
    pltpu.sync_copy(red, maxp_hbm.at[c, pl.ds(g0, 8)])
    pltpu.sync_copy(gred, gmaxp_hbm.at[c, pl.ds(g0, 8)])


def _sc_pool(h3, gate, batch_p):
    mesh = plsc.VectorSubcoreMesh(core_axis_name="c", subcore_axis_name="s")
    f = pl.kernel(
        _pool_body,
        out_type=(jax.ShapeDtypeStruct((NC, G, H), jnp.float32),
                  jax.ShapeDtypeStruct((NC, G, LANES), jnp.float32)),
        mesh=mesh,
        compiler_params=pltpu.CompilerParams(needs_layout_passes=False),
        scratch_types=[
            pltpu.VMEM((TROWS, H), jnp.float32),
            pltpu.VMEM((TROWS, LANES), jnp.float32),
            pltpu.VMEM((144, H), jnp.float32),
            pltpu.VMEM((144,), jnp.float32),
            pltpu.VMEM((144,), jnp.int32),
            pltpu.VMEM((8, H), jnp.float32),
            pltpu.VMEM((8, LANES), jnp.float32),
            pltpu.VMEM((8, LANES), jnp.float32),
            pltpu.VMEM_SHARED((NS, TROWS, H), jnp.float32),
            pltpu.VMEM_SHARED((NS, TROWS, LANES), jnp.float32),
        ],
    )
    return f(h3, gate, batch_p)


def _full(shape):
    return pl.BlockSpec(shape, lambda i: (0,) * len(shape))


def _rows(din):
    return pl.BlockSpec((B, din), lambda i: (i, 0))


def _b3spec():
    return pl.BlockSpec((1, 1, B), lambda i: (i, 0, 0))


def _layer_body(din, residual, pool, args):
    if pool:
        (hin, agg, b3, w1t, b1, g1, be1, w2t, b2, g2, be2, gn, ben,
         a1t, ab1, a2t, ab2, hout, vt, cntp, gate3) = args
    else:
        (hin, agg, b3, w1t, b1, g1, be1, w2t, b2, g2, be2, gn, ben,
         hout, vt) = args
    z = hin[...] + agg[...]
    t = jax.nn.relu(_ln(z @ w1t[...] + b1[...], g1[...], be1[...]))
    t = jax.nn.relu(_ln(t @ w2t[...] + b2[...], g2[...], be2[...]))
    h2 = jax.nn.relu(_ln(t, gn[...], ben[...]))
    if residual:
        h2 = h2 + hin[...]
    hout[...] = h2
    bt = b3[0, 0, :]
    gid = lax.broadcasted_iota(jnp.int32, (G, B), 0)
    oh = (gid == bt[None, :]).astype(jnp.float32)
    part = jnp.dot(oh, h2, preferred_element_type=jnp.float32)
    pid = pl.program_id(0)

    @pl.when(pid == 0)
    def _():
        vt[...] = part

    @pl.when(pid != 0)
    def _():
        vt[...] += part

    if pool:
        cpart = jnp.dot(oh, jnp.ones((B, H), jnp.float32),
                        preferred_element_type=jnp.float32)

        @pl.when(pid == 0)
        def _():
            cntp[...] = cpart

        @pl.when(pid != 0)
        def _():
            cntp[...] += cpart

        t1 = jax.nn.relu(h2 @ a1t[...] + ab1[...])
        gv = jnp.dot(t1, a2t[...], preferred_element_type=jnp.float32) + ab2[...]
        gate3[0, 0, :] = gv[:, 0]


def _layer_tc(hin, agg, b3, p, i, residual, pool):
    din = hin.shape[1]
    w1t = p['conv%d_mlp1_W' % i].T
    if w1t.shape[0] < din:
        w1t = jnp.zeros((din, H), jnp.float32).at[:w1t.shape[0]].set(w1t)
    ws = [w1t, p['conv%d_mlp1_b' % i][None],
          p['conv%d_mlp1_g' % i][None], p['conv%d_mlp1_be' % i][None],
          p['conv%d_mlp2_W' % i].T, p['conv%d_mlp2_b' % i][None],
          p['conv%d_mlp2_g' % i][None], p['conv%d_mlp2_be' % i][None],
          p['norm%d_g' % i][None], p['norm%d_be' % i][None]]
    in_specs = [_rows(din), _rows(din), _b3spec(),
                _full((din, H)), _full((1, H)), _full((1, H)), _full((1, H)),
                _full((H, H)), _full((1, H)), _full((1, H)), _full((1, H)),
                _full((1, H)), _full((1, H))]
    out_shape = [jax.ShapeDtypeStruct((NPAD, H), jnp.float32),
                 jax.ShapeDtypeStruct((G, H), jnp.float32)]
    out_specs = [_rows(H), _full((G, H))]
    args = [hin, agg, b3] + ws
    if pool:
        args += [p['att1_W'].T, p['att1_b'][None],
                 p['att2_W'].T, p['att2_b'][None]]
        in_specs += [_full((H, H)), _full((1, H)), _full((H, 1)),
                     _full((1, 1))]
        out_shape += [jax.ShapeDtypeStruct((G, H), jnp.float32),
                      jax.ShapeDtypeStruct((NB, 1, B), jnp.float32)]
        out_specs += [_full((G, H)), _b3spec()]
    body = lambda *a: _layer_body(din, residual, pool, a)
    return pl.pallas_call(
        body, grid=(NB,), in_specs=in_specs, out_specs=out_specs,
        out_shape=out_shape)(*args)


def _vn_body(args):
    vn, vt, w1t, b1, g1, be1, w2t, b2, g2, be2, out = args
    v = vn[...] + vt[...]
    v = jax.nn.relu(_ln(v @ w1t[...] + b1[...], g1[...], be1[...]))
    out[...] = jax.nn.relu(_ln(v @ w2t[...] + b2[...], g2[...], be2[...]))


def _vn_tc(vn, vt, p, j):
    args = [vn, vt,
            p['vn%d_1_W' % j].T, p['vn%d_1_b' % j][None],
            p['vn%d_1_g' % j][None], p['vn%d_1_be' % j][None],
            p['vn%d_2_W' % j].T, p['vn%d_2_b' % j][None],
            p['vn%d_2_g' % j][None], p['vn%d_2_be' % j][None]]
    return pl.pallas_call(
        lambda *a: _vn_body(a),
        out_shape=jax.ShapeDtypeStruct((G, H), jnp.float32))(*args)


def _vnadd_body(hin, b3, vn, hout):
    bt = b3[0, 0, :]
    gid = lax.broadcasted_iota(jnp.int32, (B, G), 1)
    oh = (bt[:, None] == gid).astype(jnp.float32)
    hout[...] = hin[...] + jnp.dot(oh, vn[...],
                                   preferred_element_type=jnp.float32)


def _vnadd_tc(hin, b3, vn):
    return pl.pallas_call(
        _vnadd_body, grid=(NB,),
        in_specs=[_rows(H), _b3spec(), _full((G, H))],
        out_specs=_rows(H),
        out_shape=jax.ShapeDtypeStruct((NPAD, H), jnp.float32))(hin, b3, vn)


def _att_body(h3, gate3, b3, gmaxp, attsum, den):
    gm = jnp.maximum(gmaxp[0], gmaxp[1])
    bt = b3[0, 0, :]
    gidb = lax.broadcasted_iota(jnp.int32, (B, G), 1)
    ohb = (bt[:, None] == gidb).astype(jnp.float32)
    gmb = jnp.dot(ohb, gm, preferred_element_type=jnp.float32)
    eg = jnp.exp(gate3[0, 0, :][:, None] - gmb)
    gid = lax.broadcasted_iota(jnp.int32, (G, B), 0)
    oh = (gid == bt[None, :]).astype(jnp.float32)
    apart = jnp.dot(oh, eg[:, 0:1] * h3[...],
                    preferred_element_type=jnp.float32)
    dpart = jnp.dot(oh, eg, preferred_element_type=jnp.float32)
    pid = pl.program_id(0)

    @pl.when(pid == 0)
    def _():
        attsum[...] = apart
        den[...] = dpart

    @pl.when(pid != 0)
    def _():
        attsum[...] += apart
        den[...] += dpart


def _att_tc(h3, gate3, b3, gmaxp):
    return pl.pallas_call(
        _att_body, grid=(NB,),
        in_specs=[_rows(H), _b3spec(), _b3spec(),
                  _full((NC, G, LANES))],
        out_specs=[_full((G, H)), _full((G, LANES))],
        out_shape=[jax.ShapeDtypeStruct((G, H), jnp.float32),
                   jax.ShapeDtypeStruct((G, LANES), jnp.float32)])(
            h3, gate3, b3, gmaxp)


def _cls_body(args):
    (sm, cnt, maxp, attsum, den, w1t, b1, g1, be1, w2t, b2, g2, be2,
     w3t, b3w, out) = args
    x_add = sm[...]
    cw = jnp.maximum(cnt[...], 1.0)
    x_mean = x_add / cw
    x_max = jnp.maximum(maxp[0], maxp[1])
    x_att = attsum[...] / den[:, 0:1]
    pooled = jnp.concatenate([x_mean, x_add, x_max, x_att], axis=1)
    o = jax.nn.relu(_ln(pooled @ w1t[...] + b1[...], g1[...], be1[...]))
    o = jax.nn.relu(_ln(o @ w2t[...] + b2[...], g2[...], be2[...]))
    out[...] = o @ w3t[...] + b3w[...]


def _cls_tc(sm, cnt, maxp, attsum, den, p):
    args = [sm, cnt, maxp, attsum, den,
            p['cls1_W'].T, p['cls1_b'][None], p['cls1_g'][None],
            p['cls1_be'][None],
            p['cls2_W'].T, p['cls2_b'][None], p['cls2_g'][None],
            p['cls2_be'][None],
            p['cls3_W'].T, p['cls3_b'][None]]
    return pl.pallas_call(
        lambda *a: _cls_body(a),
        out_shape=jax.ShapeDtypeStruct((G, NUM_TASKS), jnp.float32))(*args)


def kernel(x, edge_index, edge_attr, batch, params):
    p = params
    N = x.shape[0]
    E = edge_index.shape[1]

    xpad = jnp.zeros((NPAD, H), jnp.float32).at[:N, :9].set(x)
    batch_p = jnp.full((NPAD,), G, jnp.int32).at[:N].set(batch)
    b3 = batch_p.reshape(NB, 1, B)

    EP = ((E + NS * CB - 1) // (NS * CB)) * (NS * CB)
    srcp = jnp.zeros((EP,), jnp.int32).at[:E].set(edge_index[0])
    dstp = jnp.full((EP,), BIGDST, jnp.int32).at[:E].set(edge_index[1])
    a0p = jnp.zeros((EP,), jnp.float32).at[:E].set(edge_attr[:, 0])
    a1p = jnp.zeros((EP,), jnp.float32).at[:E].set(edge_attr[:, 1])
    a2p = jnp.zeros((EP,), jnp.float32).at[:E].set(edge_attr[:, 2])

    wt0 = jnp.zeros((3, H), jnp.float32).at[:, :9].set(p['conv0_edge_W'].T)
    be0 = jnp.zeros((H,), jnp.float32).at[:9].set(p['conv0_edge_b'])

    vn = jnp.broadcast_to(p['vn_table'][0:1], (G, H))

    agg0 = _mp(xpad, srcp, dstp, a0p, a1p, a2p, wt0, be0, H, 3)
    h1, vt0 = _layer_tc(xpad, agg0, b3, p, 0, False, False)
    vn1 = _vn_tc(vn, vt0, p, 0)
    h1p = _vnadd_tc(h1, b3, vn1)

    agg1 = _mp(h1p, srcp, dstp, a0p, a1p, a2p,
               p['conv1_edge_W'].T, p['conv1_edge_b'], H, 3)
    h2, vt1 = _layer_tc(h1p, agg1, b3, p, 1, True, False)
    vn2 = _vn_tc(vn1, vt1, p, 1)
    h2p = _vnadd_tc(h2, b3, vn2)

    agg2 = _mp(h2p, srcp, dstp, a0p, a1p, a2p,
               p['conv2_edge_W'].T, p['conv2_edge_b'], H, 3)
    h3, sum3, cnt3, gate3 = _layer_tc(h2p, agg2, b3, p, 2, True, True)
    maxp, gmaxp = _sc_pool(h3, gate3.reshape(NPAD), batch_p)
    attsum, den = _att_tc(h3, gate3, b3, gmaxp)
    return _cls_tc(sum3, cnt3, maxp, attsum, den, p)
